# baseline (speedup 1.0000x reference)
import functools

import jax
import jax.numpy as jnp
from jax import lax
from jax.experimental import pallas as pl
from jax.experimental.pallas import tpu as pltpu
from jax.experimental.pallas import tpu_sc as plsc

N_EXPERTS = 64
TOPK = 8
LANES = 16
NUM_WORKERS = 32
ROW_BLOCK = 512


def _logits_body(w_ref, hs_ref, out_ref):
    out_ref[...] = lax.dot_general(
        w_ref[...], hs_ref[...],
        dimension_numbers=(((1,), (1,)), ((), ())),
        preferred_element_type=jnp.float32)


def _compute_logits_t(hs, w, chunk, row0):
    n, h = hs.shape
    blocks = chunk // ROW_BLOCK
    base = row0 // ROW_BLOCK
    return pl.pallas_call(
        _logits_body,
        grid=(blocks,),
        in_specs=[
            pl.BlockSpec((N_EXPERTS, h), lambda i: (0, 0)),
            pl.BlockSpec((ROW_BLOCK, h), lambda i: (base + i, 0)),
        ],
        out_specs=pl.BlockSpec((N_EXPERTS, ROW_BLOCK), lambda i: (0, i)),
        out_shape=jax.ShapeDtypeStruct((N_EXPERTS, chunk), jnp.float32),
        compiler_params=pltpu.CompilerParams(
            dimension_semantics=("parallel",)),
    )(w, hs)


def _comb(a, b):
    av, ai = a
    bv, bi = b
    take_a = av >= bv
    return jnp.where(take_a, av, bv), jnp.where(take_a, ai, bi)


def _tree_max(pairs):
    while len(pairs) > 1:
        nxt = [_comb(pairs[k], pairs[k + 1])
               for k in range(0, len(pairs) - 1, 2)]
        if len(pairs) % 2:
            nxt.append(pairs[-1])
        pairs = nxt
    return pairs[0]


def _make_route(n_rows):
    rows_per_w = n_rows // NUM_WORKERS
    groups = rows_per_w // LANES
    mesh = plsc.VectorSubcoreMesh(core_axis_name="c", subcore_axis_name="s")

    @functools.partial(
        pl.kernel,
        out_type=[
            jax.ShapeDtypeStruct((n_rows * TOPK,), jnp.float32),
            jax.ShapeDtypeStruct((n_rows * TOPK,), jnp.int32),
        ],
        mesh=mesh,
        scratch_types=[
            pltpu.VMEM((N_EXPERTS, rows_per_w), jnp.float32),
            pltpu.VMEM((rows_per_w * TOPK,), jnp.float32),
            pltpu.VMEM((rows_per_w * TOPK,), jnp.int32),
        ],
        compiler_params=pltpu.CompilerParams(needs_layout_passes=False),
    )
    def route(logits_hbm, w_out, e_out, slab, w_buf, e_buf):
        wid = lax.axis_index("s") * 2 + lax.axis_index("c")
        base = wid * rows_per_w
        pltpu.sync_copy(logits_hbm.at[:, pl.ds(base, rows_per_w)], slab)

        n_blk = N_EXPERTS // TOPK
        neg_inf = jnp.full((LANES,), -jnp.inf, jnp.float32)

        def group_body(g, carry):
            rows = g * LANES + lax.iota(jnp.int32, LANES)
            bm = []
            bi = []
            for b in range(n_blk):
                leaves = []
                for e in range(b * 8, b * 8 + 8):
                    col = jnp.full((LANES,), e, jnp.int32)
                    v = slab[e, pl.ds(g * LANES, LANES)]
                    leaves.append((v, col))
                v, i = _tree_max(leaves)
                bm.append(v)
                bi.append(i)
            sel_v = []
            sel_i = []
            for j in range(TOPK):
                gv, gi = _tree_max(list(zip(bm, bi)))
                sel_v.append(gv)
                sel_i.append(gi)
                if j == TOPK - 1:
                    break
                plsc.store_scatter(slab, [gi, rows], neg_inf)
                blk8 = gi & ~jnp.int32(7)
                leaves = []
                for t in range(8):
                    idx = blk8 + t
                    v = plsc.load_gather(slab, [idx, rows])
                    leaves.append((v, idx))
                nv, ni = _tree_max(leaves)
                for b in range(n_blk):
                    is_b = blk8 == (b * 8)
                    bm[b] = jnp.where(is_b, nv, bm[b])
                    bi[b] = jnp.where(is_b, ni, bi[b])
            es = [jnp.exp(v - sel_v[0]) for v in sel_v]
            s = es[0]
            for j in range(1, TOPK):
                s = s + es[j]
            rcp = 1.0 / s
            out_off = rows * TOPK
            for j in range(TOPK):
                plsc.store_scatter(w_buf, [out_off + j], es[j] * rcp)
                plsc.store_scatter(e_buf, [out_off + j], sel_i[j])
            return carry

        lax.fori_loop(0, groups, group_body, 0)
        pltpu.sync_copy(w_buf,
                        w_out.at[pl.ds(base * TOPK, rows_per_w * TOPK)])
        pltpu.sync_copy(e_buf,
                        e_out.at[pl.ds(base * TOPK, rows_per_w * TOPK)])

    return route


CHUNK_SIZES = (16384,)


@jax.jit
def kernel(hidden_states, W):
    routes = {c: _make_route(c) for c in set(CHUNK_SIZES)}
    ws = []
    es = []
    row0 = 0
    for chunk in CHUNK_SIZES:
        logits_t = _compute_logits_t(hidden_states, W, chunk, row0)
        w_c, e_c = routes[chunk](logits_t)
        ws.append(w_c.reshape(chunk, TOPK))
        es.append(e_c.reshape(chunk, TOPK))
        row0 += chunk
    return jnp.concatenate(ws, 0), jnp.concatenate(es, 0)


# speedup vs baseline: 1.0919x; 1.0919x over previous
import functools

import jax
import jax.numpy as jnp
from jax import lax
from jax.experimental import pallas as pl
from jax.experimental.pallas import tpu as pltpu
from jax.experimental.pallas import tpu_sc as plsc

N_EXPERTS = 64
TOPK = 8
LANES = 16
NUM_WORKERS = 32
ROW_BLOCK = 512


def _logits_body(w_ref, hs_ref, out_ref):
    out_ref[...] = lax.dot_general(
        w_ref[...], hs_ref[...],
        dimension_numbers=(((1,), (1,)), ((), ())),
        preferred_element_type=jnp.float32)


def _compute_logits_t(hs, w, chunk, row0):
    n, h = hs.shape
    blocks = chunk // ROW_BLOCK
    base = row0 // ROW_BLOCK
    return pl.pallas_call(
        _logits_body,
        grid=(blocks,),
        in_specs=[
            pl.BlockSpec((N_EXPERTS, h), lambda i: (0, 0)),
            pl.BlockSpec((ROW_BLOCK, h), lambda i: (base + i, 0)),
        ],
        out_specs=pl.BlockSpec((N_EXPERTS, ROW_BLOCK), lambda i: (0, i)),
        out_shape=jax.ShapeDtypeStruct((N_EXPERTS, chunk), jnp.float32),
        compiler_params=pltpu.CompilerParams(
            dimension_semantics=("parallel",)),
    )(w, hs)


def _comb(a, b):
    av, ai = a
    bv, bi = b
    take_a = av >= bv
    return jnp.where(take_a, av, bv), jnp.where(take_a, ai, bi)


def _tree_max(pairs):
    while len(pairs) > 1:
        nxt = [_comb(pairs[k], pairs[k + 1])
               for k in range(0, len(pairs) - 1, 2)]
        if len(pairs) % 2:
            nxt.append(pairs[-1])
        pairs = nxt
    return pairs[0]


def _make_route(n_rows):
    rows_per_w = n_rows // NUM_WORKERS
    groups = rows_per_w // LANES
    mesh = plsc.VectorSubcoreMesh(core_axis_name="c", subcore_axis_name="s")

    @functools.partial(
        pl.kernel,
        out_type=[
            jax.ShapeDtypeStruct((n_rows * TOPK,), jnp.float32),
            jax.ShapeDtypeStruct((n_rows * TOPK,), jnp.int32),
        ],
        mesh=mesh,
        scratch_types=[
            pltpu.VMEM((N_EXPERTS, rows_per_w), jnp.float32),
            pltpu.VMEM((rows_per_w * TOPK,), jnp.float32),
            pltpu.VMEM((rows_per_w * TOPK,), jnp.int32),
        ],
        compiler_params=pltpu.CompilerParams(needs_layout_passes=False),
    )
    def route(logits_hbm, w_out, e_out, slab, w_buf, e_buf):
        wid = lax.axis_index("s") * 2 + lax.axis_index("c")
        base = wid * rows_per_w
        pltpu.sync_copy(logits_hbm.at[:, pl.ds(base, rows_per_w)], slab)

        n_blk = N_EXPERTS // TOPK
        neg_inf = jnp.full((LANES,), -jnp.inf, jnp.float32)

        def group_body(g, carry):
            rows = g * LANES + lax.iota(jnp.int32, LANES)
            bm = []
            bi = []
            for b in range(n_blk):
                leaves = []
                for e in range(b * 8, b * 8 + 8):
                    col = jnp.full((LANES,), e, jnp.int32)
                    v = slab[e, pl.ds(g * LANES, LANES)]
                    leaves.append((v, col))
                v, i = _tree_max(leaves)
                bm.append(v)
                bi.append(i)
            sel_v = []
            sel_i = []
            for j in range(TOPK):
                gv, gi = _tree_max(list(zip(bm, bi)))
                sel_v.append(gv)
                sel_i.append(gi)
                if j == TOPK - 1:
                    break
                plsc.store_scatter(slab, [gi, rows], neg_inf)
                blk8 = gi & ~jnp.int32(7)
                leaves = []
                for t in range(8):
                    idx = blk8 + t
                    v = plsc.load_gather(slab, [idx, rows])
                    leaves.append((v, idx))
                nv, ni = _tree_max(leaves)
                for b in range(n_blk):
                    is_b = blk8 == (b * 8)
                    bm[b] = jnp.where(is_b, nv, bm[b])
                    bi[b] = jnp.where(is_b, ni, bi[b])
            es = [jnp.exp(v - sel_v[0]) for v in sel_v]
            s = es[0]
            for j in range(1, TOPK):
                s = s + es[j]
            rcp = 1.0 / s
            out_off = rows * TOPK
            for j in range(TOPK):
                plsc.store_scatter(w_buf, [out_off + j], es[j] * rcp)
                plsc.store_scatter(e_buf, [out_off + j], sel_i[j])
            return carry

        lax.fori_loop(0, groups, group_body, 0)
        pltpu.sync_copy(w_buf,
                        w_out.at[pl.ds(base * TOPK, rows_per_w * TOPK)])
        pltpu.sync_copy(e_buf,
                        e_out.at[pl.ds(base * TOPK, rows_per_w * TOPK)])

    return route


CHUNK_SIZES = (8192, 8192)


@jax.jit
def kernel(hidden_states, W):
    routes = {c: _make_route(c) for c in set(CHUNK_SIZES)}
    logits = []
    row0 = 0
    for chunk in CHUNK_SIZES:
        logits.append(_compute_logits_t(hidden_states, W, chunk, row0))
        row0 += chunk
    ws = []
    es = []
    for chunk, logits_t in zip(CHUNK_SIZES, logits):
        w_c, e_c = routes[chunk](logits_t)
        ws.append(w_c.reshape(chunk, TOPK))
        es.append(e_c.reshape(chunk, TOPK))
    return jnp.concatenate(ws, 0), jnp.concatenate(es, 0)
